# baseline (speedup 1.0000x reference)
import functools

import jax
import jax.numpy as jnp
from jax.experimental import pallas as pl
from jax.experimental.pallas import tpu as pltpu


def _mp_kernel(x_ref, a_ref, w0_ref, w1_ref, out_ref, ht_ref):
    l = pl.program_id(0)
    f = pl.program_id(1)

    @pl.when(jnp.logical_and(l == 0, f == 0))
    def _init_h():
        ht_ref[...] = jax.lax.dot_general(
            w0_ref[...], x_ref[...],
            dimension_numbers=(((0,), (1,)), ((), ())),
            preferred_element_type=jnp.float32)

    @pl.when(jnp.logical_and(l == 1, f == 0))
    def _next_h():
        ht_ref[...] = jax.lax.dot_general(
            w1_ref[...], out_ref[...],
            dimension_numbers=(((0,), (0,)), ((), ())),
            preferred_element_type=jnp.float32)

    hrow = ht_ref[pl.ds(f, 1), :]
    a = a_ref[0]
    row = jax.lax.dot_general(
        hrow, a, dimension_numbers=(((1,), (1,)), ((), ())),
        preferred_element_type=jnp.float32)
    out_ref[pl.ds(f, 1), :] = jnp.maximum(row, 0.0)


def _message_pass(x, A, W0, W1):
    F, N, _ = A.shape
    out_t = pl.pallas_call(
        _mp_kernel,
        grid=(2, F),
        in_specs=[
            pl.BlockSpec(x.shape, lambda l, f: (0, 0)),
            pl.BlockSpec((1, N, N), lambda l, f: (f, 0, 0)),
            pl.BlockSpec(W0.shape, lambda l, f: (0, 0)),
            pl.BlockSpec(W1.shape, lambda l, f: (0, 0)),
        ],
        out_specs=pl.BlockSpec((F, N), lambda l, f: (0, 0)),
        out_shape=jax.ShapeDtypeStruct((F, N), jnp.float32),
        scratch_shapes=[
            pltpu.VMEM((F, N), jnp.float32),
        ],
    )(x, A, W0, W1)
    return out_t.T


@jax.jit
def kernel(node_x, edge_x, node_adjacency_tensor, edge_adjacency_tensor,
           node_W0, node_W1, edge_W0, edge_W1):
    node_out = _message_pass(node_x, node_adjacency_tensor, node_W0, node_W1)
    edge_out = _message_pass(edge_x, edge_adjacency_tensor, edge_W0, edge_W1)
    return (node_out, edge_out)


# speedup vs baseline: 1.4307x; 1.4307x over previous
import functools

import jax
import jax.numpy as jnp
from jax.experimental import pallas as pl
from jax.experimental.pallas import tpu as pltpu


def _mp_kernel(C, x_ref, a_ref, w0_ref, w1_ref, out_ref, ht_ref):
    l = pl.program_id(0)
    k = pl.program_id(1)

    @pl.when(jnp.logical_and(l == 0, k == 0))
    def _init_h():
        ht_ref[...] = jax.lax.dot_general(
            w0_ref[...], x_ref[...],
            dimension_numbers=(((0,), (1,)), ((), ())),
            preferred_element_type=jnp.float32)

    @pl.when(jnp.logical_and(l == 1, k == 0))
    def _next_h():
        ht_ref[...] = jax.lax.dot_general(
            w1_ref[...], out_ref[...],
            dimension_numbers=(((0,), (0,)), ((), ())),
            preferred_element_type=jnp.float32)

    for c in range(C):
        f = k * C + c
        hrow = ht_ref[pl.ds(f, 1), :]
        a = a_ref[c]
        row = jax.lax.dot_general(
            hrow, a, dimension_numbers=(((1,), (1,)), ((), ())),
            preferred_element_type=jnp.float32)
        out_ref[pl.ds(f, 1), :] = jnp.maximum(row, 0.0)


def _message_pass(x, A, W0, W1, C):
    F, N, _ = A.shape
    out_t = pl.pallas_call(
        functools.partial(_mp_kernel, C),
        grid=(2, F // C),
        in_specs=[
            pl.BlockSpec(x.shape, lambda l, k: (0, 0)),
            pl.BlockSpec((C, N, N), lambda l, k: (k, 0, 0)),
            pl.BlockSpec(W0.shape, lambda l, k: (0, 0)),
            pl.BlockSpec(W1.shape, lambda l, k: (0, 0)),
        ],
        out_specs=pl.BlockSpec((F, N), lambda l, k: (0, 0)),
        out_shape=jax.ShapeDtypeStruct((F, N), jnp.float32),
        scratch_shapes=[
            pltpu.VMEM((F, N), jnp.float32),
        ],
    )(x, A, W0, W1)
    return out_t.T


@jax.jit
def kernel(node_x, edge_x, node_adjacency_tensor, edge_adjacency_tensor,
           node_W0, node_W1, edge_W0, edge_W1):
    node_out = _message_pass(node_x, node_adjacency_tensor, node_W0, node_W1, C=8)
    edge_out = _message_pass(edge_x, edge_adjacency_tensor, edge_W0, edge_W1, C=2)
    return (node_out, edge_out)


# speedup vs baseline: 1.4405x; 1.0069x over previous
import functools

import jax
import jax.numpy as jnp
from jax.experimental import pallas as pl
from jax.experimental.pallas import tpu as pltpu


def _mp_kernel(C, x_ref, a_ref, w0_ref, w1_ref, out_ref, ht_ref):
    l = pl.program_id(0)
    k = pl.program_id(1)

    @pl.when(jnp.logical_and(l == 0, k == 0))
    def _init_h():
        ht_ref[...] = jax.lax.dot_general(
            w0_ref[...], x_ref[...],
            dimension_numbers=(((0,), (1,)), ((), ())),
            preferred_element_type=jnp.float32)

    @pl.when(jnp.logical_and(l == 1, k == 0))
    def _next_h():
        ht_ref[...] = jax.lax.dot_general(
            w1_ref[...], out_ref[...],
            dimension_numbers=(((0,), (0,)), ((), ())),
            preferred_element_type=jnp.float32)

    for c in range(C):
        f = k * C + c
        hrow = ht_ref[pl.ds(f, 1), :]
        a = a_ref[c]
        row = jax.lax.dot_general(
            hrow, a, dimension_numbers=(((1,), (1,)), ((), ())),
            preferred_element_type=jnp.float32)
        out_ref[pl.ds(f, 1), :] = jnp.maximum(row, 0.0)


def _message_pass(x, A, W0, W1, C):
    F, N, _ = A.shape
    if C == F:
        a_spec = pl.BlockSpec((F, N, N), lambda l, k: (0, 0, 0))
    else:
        a_spec = pl.BlockSpec((C, N, N), lambda l, k: (k, 0, 0))
    out_t = pl.pallas_call(
        functools.partial(_mp_kernel, C),
        grid=(2, F // C),
        in_specs=[
            pl.BlockSpec(x.shape, lambda l, k: (0, 0)),
            a_spec,
            pl.BlockSpec(W0.shape, lambda l, k: (0, 0)),
            pl.BlockSpec(W1.shape, lambda l, k: (0, 0)),
        ],
        out_specs=pl.BlockSpec((F, N), lambda l, k: (0, 0)),
        out_shape=jax.ShapeDtypeStruct((F, N), jnp.float32),
        scratch_shapes=[
            pltpu.VMEM((F, N), jnp.float32),
        ],
    )(x, A, W0, W1)
    return out_t.T


@jax.jit
def kernel(node_x, edge_x, node_adjacency_tensor, edge_adjacency_tensor,
           node_W0, node_W1, edge_W0, edge_W1):
    node_out = _message_pass(node_x, node_adjacency_tensor, node_W0, node_W1, C=32)
    edge_out = _message_pass(edge_x, edge_adjacency_tensor, edge_W0, edge_W1, C=4)
    return (node_out, edge_out)
